# baseline (speedup 1.0000x reference)
import functools

import jax
import jax.numpy as jnp
from jax import lax
from jax.experimental import pallas as pl
from jax.experimental.pallas import tpu as pltpu
from jax.experimental.pallas import tpu_sc as plsc

NUM_EMB = 1000000
DIM = 64
BATCH = 4096
HIST = 50

NC = 2
NS = 16
NW = NC * NS
TOTAL = BATCH * HIST
B_PER_W = TOTAL // NW
CHUNK = 128
NCHUNK = B_PER_W // CHUNK
NBUF = 2


def _body(idx_hbm, table_hbm, out_hbm, idx_v, rows_v, gsem, osem):
  wid = lax.axis_index("s") * NC + lax.axis_index("c")
  base = wid * B_PER_W

  pltpu.sync_copy(idx_hbm.at[wid], idx_v)

  def start_gather(j, b):
    pltpu.async_copy(table_hbm.at[idx_v.at[j]], rows_v.at[b], gsem.at[b])

  def wait_gather(j, b):
    pltpu.make_async_copy(table_hbm.at[idx_v.at[j]], rows_v.at[b],
                          gsem.at[b]).wait()

  def start_write(j, b):
    pltpu.async_copy(rows_v.at[b], out_hbm.at[pl.ds(base + j * CHUNK, CHUNK)],
                     osem.at[b])

  def wait_write(j, b):
    pltpu.make_async_copy(rows_v.at[b], out_hbm.at[pl.ds(base + j * CHUNK,
                                                         CHUNK)],
                          osem.at[b]).wait()

  for b in range(NBUF):
    start_gather(b, b)

  @pl.loop(0, NCHUNK, step=NBUF)
  def _outer(j0):
    for b in range(NBUF):
      j = j0 + b
      wait_gather(j, b)
      start_write(j, b)
      wait_write(j, b)

      @pl.when(j + NBUF < NCHUNK)
      def _():
        start_gather(j + NBUF, b)


@jax.jit
def kernel(indices, table):
  idx = indices.reshape(NW, NCHUNK, CHUNK).astype(jnp.int32)
  run = pl.kernel(
      _body,
      out_type=jax.ShapeDtypeStruct((TOTAL, DIM), jnp.float32),
      mesh=plsc.VectorSubcoreMesh(core_axis_name="c", subcore_axis_name="s"),
      compiler_params=pltpu.CompilerParams(use_tc_tiling_on_sc=False),
      scratch_types=[
          pltpu.VMEM((NCHUNK, CHUNK), jnp.int32),
          pltpu.VMEM((NBUF, CHUNK, DIM), jnp.float32),
          pltpu.SemaphoreType.DMA((NBUF,)),
          pltpu.SemaphoreType.DMA((NBUF,)),
      ],
  )
  out = run(idx, table)
  return out.reshape(BATCH, HIST, DIM)


# speedup vs baseline: 1.0093x; 1.0093x over previous
import functools

import jax
import jax.numpy as jnp
from jax import lax
from jax.experimental import pallas as pl
from jax.experimental.pallas import tpu as pltpu
from jax.experimental.pallas import tpu_sc as plsc

NUM_EMB = 1000000
DIM = 64
BATCH = 4096
HIST = 50

NC = 2
NS = 16
NW = NC * NS
TOTAL = BATCH * HIST
B_PER_W = TOTAL // NW
CHUNK = 128
NCHUNK = B_PER_W // CHUNK
NBUF = 5


def _body(idx_hbm, table_hbm, out_hbm, idx_v, rows_v, gsem, osem):
  wid = lax.axis_index("s") * NC + lax.axis_index("c")
  base = wid * B_PER_W

  pltpu.sync_copy(idx_hbm.at[wid], idx_v)

  def start_gather(j, b):
    pltpu.async_copy(table_hbm.at[idx_v.at[j]], rows_v.at[b], gsem.at[b])

  def wait_gather(j, b):
    pltpu.make_async_copy(table_hbm.at[idx_v.at[j]], rows_v.at[b],
                          gsem.at[b]).wait()

  def start_write(j, b):
    pltpu.async_copy(rows_v.at[b], out_hbm.at[pl.ds(base + j * CHUNK, CHUNK)],
                     osem.at[b])

  def wait_write(j, b):
    pltpu.make_async_copy(rows_v.at[b], out_hbm.at[pl.ds(base + j * CHUNK,
                                                         CHUNK)],
                          osem.at[b]).wait()

  for b in range(NBUF):
    start_gather(b, b)

  @pl.loop(0, NCHUNK, step=NBUF)
  def _outer(j0):
    for b in range(NBUF):
      j = j0 + b
      bp = (b - 1) % NBUF
      wait_gather(j, b)
      start_write(j, b)
      if b == 0:
        @pl.when(j >= 1)
        def _():
          wait_write(j - 1, bp)

        @pl.when((j >= 1) & (j + NBUF - 1 < NCHUNK))
        def _():
          start_gather(j + NBUF - 1, bp)
      else:
        wait_write(j - 1, bp)

        @pl.when(j + NBUF - 1 < NCHUNK)
        def _():
          start_gather(j + NBUF - 1, bp)

  wait_write(NCHUNK - 1, (NCHUNK - 1) % NBUF)


@jax.jit
def kernel(indices, table):
  idx = indices.reshape(NW, NCHUNK, CHUNK).astype(jnp.int32)
  run = pl.kernel(
      _body,
      out_type=jax.ShapeDtypeStruct((TOTAL, DIM), jnp.float32),
      mesh=plsc.VectorSubcoreMesh(core_axis_name="c", subcore_axis_name="s"),
      compiler_params=pltpu.CompilerParams(use_tc_tiling_on_sc=False),
      scratch_types=[
          pltpu.VMEM((NCHUNK, CHUNK), jnp.int32),
          pltpu.VMEM((NBUF, CHUNK, DIM), jnp.float32),
          pltpu.SemaphoreType.DMA((NBUF,)),
          pltpu.SemaphoreType.DMA((NBUF,)),
      ],
  )
  out = run(idx, table)
  return out.reshape(BATCH, HIST, DIM)
